# baseline (speedup 1.0000x reference)
import functools

import jax
import jax.numpy as jnp
from jax import lax
from jax.experimental import pallas as pl
from jax.experimental.pallas import tpu as pltpu
from jax.experimental.pallas import tpu_sc as plsc

B, N, L, C, FF, K = 4, 4096, 4096, 128, 512, 8
RADIUS = 0.4

RPRE = 512
QB = 256
QA = 256
NC, NS = 2, 16
NW = NC * NS
GW = 3 * C
NIDX = B * N * K
CHUNK = 128


def _pre_body(qx_ref, qf_ref, kx_ref, kf_ref, wp1_ref, bp1_ref, wd_ref,
              wa_ref, ba_ref, ws_ref, wl_ref, g_ref, pq_ref, ad2_ref):
    bf = jnp.bfloat16
    f32 = jnp.float32
    qx = qx_ref[0]
    kx = kx_ref[0]
    qf = qf_ref[0].astype(bf)
    kf = kf_ref[0].astype(bf)
    wp1 = wp1_ref[...]
    wa = wa_ref[...].astype(bf)
    pkv = (kx[:, 0:1] * wp1[0:1, :] + kx[:, 1:2] * wp1[1:2, :]
           + kx[:, 2:3] * wp1[2:3, :])
    pq = (qx[:, 0:1] * wp1[0:1, :] + qx[:, 1:2] * wp1[1:2, :]
          + qx[:, 2:3] * wp1[2:3, :]) + bp1_ref[...]
    dot = functools.partial(jnp.dot, preferred_element_type=f32)
    t = dot(dot(kf, ws_ref[...].astype(bf)).astype(bf), wa)
    xj = dot(kf, wl_ref[...].astype(bf))
    ad2 = dot(dot(qf, wd_ref[...].astype(bf)).astype(bf), wa) + ba_ref[...]
    g_ref[0, :, 0:C] = pkv
    g_ref[0, :, C:2 * C] = t
    g_ref[0, :, 2 * C:3 * C] = xj
    pq_ref[0] = pq
    ad2_ref[0] = ad2


def _knn_body(qx_ref, kxt_ref, idx_ref, msk_ref):
    bf = jnp.bfloat16
    f32 = jnp.float32
    kx = kxt_ref[0]
    qx = qx_ref[0]
    kx0 = kx[0:1, :]
    kx1 = kx[1:2, :]
    kx2 = kx[2:3, :]
    kk = kx0 * kx0 + kx1 * kx1 + kx2 * kx2
    q0 = qx[:, 0:1]
    q1 = qx[:, 1:2]
    q2 = qx[:, 2:3]
    qq = q0 * q0 + q1 * q1 + q2 * q2
    e = (q0.astype(bf).astype(f32) * kx0.astype(bf).astype(f32)
         + q1.astype(bf).astype(f32) * kx1.astype(bf).astype(f32)
         + q2.astype(bf).astype(f32) * kx2.astype(bf).astype(f32))
    s = (qq + kk) - 2.0 * e
    iota = lax.broadcasted_iota(jnp.int32, (1, L), 1).astype(jnp.float32)
    base = (pl.program_id(0) * L).astype(jnp.float32)
    r2 = jnp.float32(RADIUS * RADIUS + 1e-6)
    for k in range(K):
        m = jnp.min(s, axis=1, keepdims=True)
        idxk = jnp.min(jnp.where(s == m, iota, jnp.float32(1e9)),
                       axis=1, keepdims=True)
        s = jnp.where(iota == idxk, jnp.float32(jnp.inf), s)
        idx_ref[0, :, k:k + 1] = (idxk + base).astype(jnp.int32)
        msk_ref[0, :, k:k + 1] = jnp.where(m <= r2, 1.0, 0.0)


def _gather_body(g_hbm, idx_hbm, out_hbm, idx_v, rows0, rows1, sem0, sem1):
    wid = lax.axis_index("s") * NC + lax.axis_index("c")
    nch = idx_hbm.shape[0] // NW
    rbase = wid * nch
    pltpu.sync_copy(idx_hbm.at[pl.ds(rbase, nch)], idx_v)
    bufs = (rows0, rows1)
    sems = (sem0, sem1)
    copies = []
    for c in range(nch):
        copies.append(
            pltpu.async_copy(g_hbm.at[idx_v.at[c]], bufs[c & 1], sems[c & 1]))
        if c >= 1:
            copies[c - 1].wait()
            pltpu.sync_copy(
                bufs[(c - 1) & 1],
                out_hbm.at[pl.ds((rbase + c - 1) * CHUNK, CHUNK)])
    copies[nch - 1].wait()
    pltpu.sync_copy(bufs[(nch - 1) & 1],
                    out_hbm.at[pl.ds((rbase + nch - 1) * CHUNK, CHUNK)])


def _attn_body(gk_ref, pq_ref, ad2_ref, msk_ref, qf_ref, wp2_ref, bp2_ref,
               wa_ref, wf1_ref, bf1_ref, wf2_ref, bf2_ref, ln1g_ref,
               ln1b_ref, ln2g_ref, ln2b_ref, out_ref):
    bf = jnp.bfloat16
    f32 = jnp.float32
    dot = functools.partial(jnp.dot, preferred_element_type=f32)
    pq = pq_ref[...]
    ad2 = ad2_ref[...]
    wp2 = wp2_ref[...].astype(bf)
    bp2 = bp2_ref[...]
    wa = wa_ref[...].astype(bf)
    neg_inf = jnp.float32(-jnp.inf)
    ams = []
    vs = []
    mks = []
    for k in range(K):
        g = gk_ref[k]
        pkv = g[:, 0:C]
        t = g[:, C:2 * C]
        xj = g[:, 2 * C:3 * C]
        h1 = jnp.maximum(pq - pkv, 0.0)
        delta = jnp.maximum(dot(h1.astype(bf), wp2) + bp2, 0.0)
        ak = jnp.maximum(ad2 - t + dot(delta.astype(bf), wa), 0.0)
        mk = msk_ref[:, k:k + 1] > 0.0
        ams.append(jnp.where(mk, ak, neg_inf))
        vs.append(xj + delta)
        mks.append(mk)
    amax = ams[0]
    for k in range(1, K):
        amax = jnp.maximum(amax, ams[k])
    amax = jnp.where(amax == neg_inf, 0.0, amax)
    sx = jnp.zeros_like(amax)
    conv = jnp.zeros_like(amax)
    for k in range(K):
        exk = jnp.where(mks[k], jnp.exp(ams[k] - amax), 0.0)
        sx = sx + exk
        conv = conv + exk * vs[k]
    conv = conv / jnp.maximum(sx, 1e-16)
    x1 = qf_ref[...] + conv
    mu = jnp.mean(x1, axis=1, keepdims=True)
    var = jnp.mean((x1 - mu) * (x1 - mu), axis=1, keepdims=True)
    o1 = (x1 - mu) / jnp.sqrt(var + 1e-5) * ln1g_ref[...] + ln1b_ref[...]
    h = dot(o1.astype(bf), wf1_ref[...].astype(bf)) + bf1_ref[...]
    g2 = 0.5 * h * (1.0 + lax.erf(h * jnp.float32(0.7071067811865476)))
    o2 = dot(g2.astype(bf), wf2_ref[...].astype(bf)) + bf2_ref[...]
    x2 = o1 + o2
    mu2 = jnp.mean(x2, axis=1, keepdims=True)
    var2 = jnp.mean((x2 - mu2) * (x2 - mu2), axis=1, keepdims=True)
    out_ref[...] = ((x2 - mu2) / jnp.sqrt(var2 + 1e-5) * ln2g_ref[...]
                    + ln2b_ref[...])


def _row2(x):
    return x.reshape(1, -1)


def kernel(q_xyz, q_feat, kv_xyz, kv_feat, W_pos1, b_pos1, W_pos2, b_pos2,
           W_attn, b_attn, W_lin, W_src, W_dst, ln1_g, ln1_b,
           W_ff1, b_ff1, W_ff2, b_ff2, ln2_g, ln2_b):
    f32 = jnp.float32

    wspec = pl.BlockSpec((C, C), lambda b, i: (0, 0))
    bspec = pl.BlockSpec((1, C), lambda b, i: (0, 0))
    g_tab, pq, ad2 = pl.pallas_call(
        _pre_body,
        grid=(B, N // RPRE),
        in_specs=[
            pl.BlockSpec((1, RPRE, 3), lambda b, i: (b, i, 0)),
            pl.BlockSpec((1, RPRE, C), lambda b, i: (b, i, 0)),
            pl.BlockSpec((1, RPRE, 3), lambda b, i: (b, i, 0)),
            pl.BlockSpec((1, RPRE, C), lambda b, i: (b, i, 0)),
            pl.BlockSpec((3, C), lambda b, i: (0, 0)),
            bspec, wspec, wspec, bspec, wspec, wspec,
        ],
        out_specs=[
            pl.BlockSpec((1, RPRE, GW), lambda b, i: (b, i, 0)),
            pl.BlockSpec((1, RPRE, C), lambda b, i: (b, i, 0)),
            pl.BlockSpec((1, RPRE, C), lambda b, i: (b, i, 0)),
        ],
        out_shape=[
            jax.ShapeDtypeStruct((B, L, GW), f32),
            jax.ShapeDtypeStruct((B, N, C), f32),
            jax.ShapeDtypeStruct((B, N, C), f32),
        ],
    )(q_xyz, q_feat, kv_xyz, kv_feat, W_pos1, _row2(b_pos1), W_dst,
      W_attn, _row2(b_attn), W_src, W_lin)

    kv_t = jnp.transpose(kv_xyz, (0, 2, 1))
    g_flat = g_tab.reshape(B * L, GW)
    mesh = plsc.VectorSubcoreMesh(core_axis_name="c", subcore_axis_name="s",
                                  num_cores=NC, num_subcores=NS)
    HN = N // 2
    nidx_h = B * HN * K
    wspec1 = pl.BlockSpec((C, C), lambda i: (0, 0))
    bspec1 = pl.BlockSpec((1, C), lambda i: (0, 0))

    def knn_half(h):
        qx_h = lax.slice_in_dim(q_xyz, h * HN, (h + 1) * HN, axis=1)
        return pl.pallas_call(
            _knn_body,
            grid=(B, HN // QB),
            in_specs=[
                pl.BlockSpec((1, QB, 3), lambda b, i: (b, i, 0)),
                pl.BlockSpec((1, 3, L), lambda b, i: (b, 0, 0)),
            ],
            out_specs=[
                pl.BlockSpec((1, QB, K), lambda b, i: (b, i, 0)),
                pl.BlockSpec((1, QB, K), lambda b, i: (b, i, 0)),
            ],
            out_shape=[
                jax.ShapeDtypeStruct((B, HN, K), jnp.int32),
                jax.ShapeDtypeStruct((B, HN, K), f32),
            ],
        )(qx_h, kv_t)

    def gather_half(idx_h):
        idx_kmaj = jnp.transpose(idx_h, (2, 0, 1)).reshape(
            nidx_h // CHUNK, CHUNK)
        nch = nidx_h // CHUNK // NW
        gk = pl.kernel(
            _gather_body,
            out_type=jax.ShapeDtypeStruct((nidx_h, GW), f32),
            mesh=mesh,
            scratch_types=[
                pltpu.VMEM((nch, CHUNK), jnp.int32),
                pltpu.VMEM((CHUNK, GW), f32),
                pltpu.VMEM((CHUNK, GW), f32),
                pltpu.SemaphoreType.DMA,
                pltpu.SemaphoreType.DMA,
            ],
        )(g_flat, idx_kmaj)
        return gk.reshape(K, B * HN, GW)

    def attn_half(h, gk, msk_h):
        sl = lambda a: lax.slice_in_dim(
            a, h * HN, (h + 1) * HN, axis=1).reshape(B * HN, C)
        out_h = pl.pallas_call(
            _attn_body,
            grid=(B * HN // QA,),
            in_specs=[
                pl.BlockSpec((K, QA, GW), lambda i: (0, i, 0)),
                pl.BlockSpec((QA, C), lambda i: (i, 0)),
                pl.BlockSpec((QA, C), lambda i: (i, 0)),
                pl.BlockSpec((QA, K), lambda i: (i, 0)),
                pl.BlockSpec((QA, C), lambda i: (i, 0)),
                wspec1, bspec1, wspec1,
                pl.BlockSpec((C, FF), lambda i: (0, 0)),
                pl.BlockSpec((1, FF), lambda i: (0, 0)),
                pl.BlockSpec((FF, C), lambda i: (0, 0)),
                bspec1, bspec1, bspec1, bspec1, bspec1,
            ],
            out_specs=pl.BlockSpec((QA, C), lambda i: (i, 0)),
            out_shape=jax.ShapeDtypeStruct((B * HN, C), f32),
        )(gk, sl(pq), sl(ad2), msk_h.reshape(B * HN, K), sl(q_feat),
          W_pos2, _row2(b_pos2), W_attn, W_ff1, _row2(b_ff1), W_ff2,
          _row2(b_ff2), _row2(ln1_g), _row2(ln1_b), _row2(ln2_g),
          _row2(ln2_b))
        return out_h.reshape(B, HN, C)

    idx0, msk0 = knn_half(0)
    gk0 = gather_half(idx0)
    idx1, msk1 = knn_half(1)
    gk1 = gather_half(idx1)
    out0 = attn_half(0, gk0, msk0)
    out1 = attn_half(1, gk1, msk1)
    return jnp.concatenate([out0, out1], axis=1)


# speedup vs baseline: 1.0685x; 1.0685x over previous
import functools

import jax
import jax.numpy as jnp
from jax import lax
from jax.experimental import pallas as pl
from jax.experimental.pallas import tpu as pltpu
from jax.experimental.pallas import tpu_sc as plsc

B, N, L, C, FF, K = 4, 4096, 4096, 128, 512, 8
RADIUS = 0.4

RPRE = 512
QB = 256
QA = 256
NC, NS = 2, 16
NW = NC * NS
GW = 3 * C
NIDX = B * N * K
CHUNK = 128


def _pre_body(qx_ref, qf_ref, kx_ref, kf_ref, wp1_ref, bp1_ref, wd_ref,
              wa_ref, ba_ref, ws_ref, wl_ref, g_ref, pq_ref, ad2_ref):
    bf = jnp.bfloat16
    f32 = jnp.float32
    qx = qx_ref[0]
    kx = kx_ref[0]
    qf = qf_ref[0].astype(bf)
    kf = kf_ref[0].astype(bf)
    wp1 = wp1_ref[...]
    wa = wa_ref[...].astype(bf)
    pkv = (kx[:, 0:1] * wp1[0:1, :] + kx[:, 1:2] * wp1[1:2, :]
           + kx[:, 2:3] * wp1[2:3, :])
    pq = (qx[:, 0:1] * wp1[0:1, :] + qx[:, 1:2] * wp1[1:2, :]
          + qx[:, 2:3] * wp1[2:3, :]) + bp1_ref[...]
    dot = functools.partial(jnp.dot, preferred_element_type=f32)
    t = dot(dot(kf, ws_ref[...].astype(bf)).astype(bf), wa)
    xj = dot(kf, wl_ref[...].astype(bf))
    ad2 = dot(dot(qf, wd_ref[...].astype(bf)).astype(bf), wa) + ba_ref[...]
    g_ref[0, :, 0:C] = pkv
    g_ref[0, :, C:2 * C] = t
    g_ref[0, :, 2 * C:3 * C] = xj
    pq_ref[0] = pq
    ad2_ref[0] = ad2


def _knn_body(qx_ref, kxt_ref, idx_ref, msk_ref):
    bf = jnp.bfloat16
    f32 = jnp.float32
    kx = kxt_ref[0]
    qx = qx_ref[0]
    kx0 = kx[0:1, :]
    kx1 = kx[1:2, :]
    kx2 = kx[2:3, :]
    kk = kx0 * kx0 + kx1 * kx1 + kx2 * kx2
    q0 = qx[:, 0:1]
    q1 = qx[:, 1:2]
    q2 = qx[:, 2:3]
    qq = q0 * q0 + q1 * q1 + q2 * q2
    e = jnp.dot(qx.astype(bf), kx.astype(bf),
                preferred_element_type=f32)
    s = (qq + kk) - 2.0 * e
    iota = lax.broadcasted_iota(jnp.int32, (1, L), 1).astype(jnp.float32)
    base = (pl.program_id(0) * L).astype(jnp.float32)
    r2 = jnp.float32(RADIUS * RADIUS + 1e-6)
    for k in range(K):
        m = jnp.min(s, axis=1, keepdims=True)
        eq = s == m
        idxk = jnp.min(jnp.where(eq, iota, jnp.float32(1e9)),
                       axis=1, keepdims=True)
        s = jnp.where(eq, jnp.float32(jnp.inf), s)
        idx_ref[0, :, k:k + 1] = (idxk + base).astype(jnp.int32)
        msk_ref[0, :, k:k + 1] = jnp.where(m <= r2, 1.0, 0.0)


def _gather_body(g_hbm, idx_hbm, out_hbm, idx_v, rows0, rows1, sem0, sem1):
    wid = lax.axis_index("s") * NC + lax.axis_index("c")
    nch = idx_hbm.shape[0] // NW
    rbase = wid * nch
    pltpu.sync_copy(idx_hbm.at[pl.ds(rbase, nch)], idx_v)
    bufs = (rows0, rows1)
    sems = (sem0, sem1)
    copies = []
    for c in range(nch):
        copies.append(
            pltpu.async_copy(g_hbm.at[idx_v.at[c]], bufs[c & 1], sems[c & 1]))
        if c >= 1:
            copies[c - 1].wait()
            pltpu.sync_copy(
                bufs[(c - 1) & 1],
                out_hbm.at[pl.ds((rbase + c - 1) * CHUNK, CHUNK)])
    copies[nch - 1].wait()
    pltpu.sync_copy(bufs[(nch - 1) & 1],
                    out_hbm.at[pl.ds((rbase + nch - 1) * CHUNK, CHUNK)])


def _attn_body(gk_ref, pq_ref, ad2_ref, msk_ref, qf_ref, wp2_ref, bp2_ref,
               wa_ref, wf1_ref, bf1_ref, wf2_ref, bf2_ref, ln1g_ref,
               ln1b_ref, ln2g_ref, ln2b_ref, out_ref):
    bf = jnp.bfloat16
    f32 = jnp.float32
    dot = functools.partial(jnp.dot, preferred_element_type=f32)
    pq = pq_ref[...]
    ad2 = ad2_ref[...]
    wp2 = wp2_ref[...].astype(bf)
    bp2 = bp2_ref[...]
    wa = wa_ref[...].astype(bf)
    neg_inf = jnp.float32(-jnp.inf)
    ams = []
    vs = []
    mks = []
    for k in range(K):
        g = gk_ref[k]
        pkv = g[:, 0:C]
        t = g[:, C:2 * C]
        xj = g[:, 2 * C:3 * C]
        h1 = jnp.maximum(pq - pkv, 0.0)
        delta = jnp.maximum(dot(h1.astype(bf), wp2) + bp2, 0.0)
        ak = jnp.maximum(ad2 - t + dot(delta.astype(bf), wa), 0.0)
        mk = msk_ref[:, k:k + 1] > 0.0
        ams.append(jnp.where(mk, ak, neg_inf))
        vs.append(xj + delta)
        mks.append(mk)
    amax = ams[0]
    for k in range(1, K):
        amax = jnp.maximum(amax, ams[k])
    amax = jnp.where(amax == neg_inf, 0.0, amax)
    sx = jnp.zeros_like(amax)
    conv = jnp.zeros_like(amax)
    for k in range(K):
        exk = jnp.where(mks[k], jnp.exp(ams[k] - amax), 0.0)
        sx = sx + exk
        conv = conv + exk * vs[k]
    conv = conv / jnp.maximum(sx, 1e-16)
    x1 = qf_ref[...] + conv
    mu = jnp.mean(x1, axis=1, keepdims=True)
    var = jnp.mean((x1 - mu) * (x1 - mu), axis=1, keepdims=True)
    o1 = (x1 - mu) / jnp.sqrt(var + 1e-5) * ln1g_ref[...] + ln1b_ref[...]
    h = dot(o1.astype(bf), wf1_ref[...].astype(bf)) + bf1_ref[...]
    g2 = 0.5 * h * (1.0 + lax.erf(h * jnp.float32(0.7071067811865476)))
    o2 = dot(g2.astype(bf), wf2_ref[...].astype(bf)) + bf2_ref[...]
    x2 = o1 + o2
    mu2 = jnp.mean(x2, axis=1, keepdims=True)
    var2 = jnp.mean((x2 - mu2) * (x2 - mu2), axis=1, keepdims=True)
    out_ref[...] = ((x2 - mu2) / jnp.sqrt(var2 + 1e-5) * ln2g_ref[...]
                    + ln2b_ref[...])


def _row2(x):
    return x.reshape(1, -1)


def kernel(q_xyz, q_feat, kv_xyz, kv_feat, W_pos1, b_pos1, W_pos2, b_pos2,
           W_attn, b_attn, W_lin, W_src, W_dst, ln1_g, ln1_b,
           W_ff1, b_ff1, W_ff2, b_ff2, ln2_g, ln2_b):
    f32 = jnp.float32

    wspec = pl.BlockSpec((C, C), lambda b, i: (0, 0))
    bspec = pl.BlockSpec((1, C), lambda b, i: (0, 0))
    g_tab, pq, ad2 = pl.pallas_call(
        _pre_body,
        grid=(B, N // RPRE),
        in_specs=[
            pl.BlockSpec((1, RPRE, 3), lambda b, i: (b, i, 0)),
            pl.BlockSpec((1, RPRE, C), lambda b, i: (b, i, 0)),
            pl.BlockSpec((1, RPRE, 3), lambda b, i: (b, i, 0)),
            pl.BlockSpec((1, RPRE, C), lambda b, i: (b, i, 0)),
            pl.BlockSpec((3, C), lambda b, i: (0, 0)),
            bspec, wspec, wspec, bspec, wspec, wspec,
        ],
        out_specs=[
            pl.BlockSpec((1, RPRE, GW), lambda b, i: (b, i, 0)),
            pl.BlockSpec((1, RPRE, C), lambda b, i: (b, i, 0)),
            pl.BlockSpec((1, RPRE, C), lambda b, i: (b, i, 0)),
        ],
        out_shape=[
            jax.ShapeDtypeStruct((B, L, GW), f32),
            jax.ShapeDtypeStruct((B, N, C), f32),
            jax.ShapeDtypeStruct((B, N, C), f32),
        ],
    )(q_xyz, q_feat, kv_xyz, kv_feat, W_pos1, _row2(b_pos1), W_dst,
      W_attn, _row2(b_attn), W_src, W_lin)

    kv_t = jnp.transpose(kv_xyz, (0, 2, 1))
    g_flat = g_tab.reshape(B * L, GW)
    mesh = plsc.VectorSubcoreMesh(core_axis_name="c", subcore_axis_name="s",
                                  num_cores=NC, num_subcores=NS)
    HN = N // 2
    nidx_h = B * HN * K
    wspec1 = pl.BlockSpec((C, C), lambda i: (0, 0))
    bspec1 = pl.BlockSpec((1, C), lambda i: (0, 0))

    def knn_half(h):
        qx_h = lax.slice_in_dim(q_xyz, h * HN, (h + 1) * HN, axis=1)
        return pl.pallas_call(
            _knn_body,
            grid=(B, HN // QB),
            in_specs=[
                pl.BlockSpec((1, QB, 3), lambda b, i: (b, i, 0)),
                pl.BlockSpec((1, 3, L), lambda b, i: (b, 0, 0)),
            ],
            out_specs=[
                pl.BlockSpec((1, QB, K), lambda b, i: (b, i, 0)),
                pl.BlockSpec((1, QB, K), lambda b, i: (b, i, 0)),
            ],
            out_shape=[
                jax.ShapeDtypeStruct((B, HN, K), jnp.int32),
                jax.ShapeDtypeStruct((B, HN, K), f32),
            ],
        )(qx_h, kv_t)

    def gather_half(idx_h):
        idx_kmaj = jnp.transpose(idx_h, (2, 0, 1)).reshape(
            nidx_h // CHUNK, CHUNK)
        nch = nidx_h // CHUNK // NW
        gk = pl.kernel(
            _gather_body,
            out_type=jax.ShapeDtypeStruct((nidx_h, GW), f32),
            mesh=mesh,
            scratch_types=[
                pltpu.VMEM((nch, CHUNK), jnp.int32),
                pltpu.VMEM((CHUNK, GW), f32),
                pltpu.VMEM((CHUNK, GW), f32),
                pltpu.SemaphoreType.DMA,
                pltpu.SemaphoreType.DMA,
            ],
        )(g_flat, idx_kmaj)
        return gk.reshape(K, B * HN, GW)

    def attn_half(h, gk, msk_h):
        sl = lambda a: lax.slice_in_dim(
            a, h * HN, (h + 1) * HN, axis=1).reshape(B * HN, C)
        out_h = pl.pallas_call(
            _attn_body,
            grid=(B * HN // QA,),
            in_specs=[
                pl.BlockSpec((K, QA, GW), lambda i: (0, i, 0)),
                pl.BlockSpec((QA, C), lambda i: (i, 0)),
                pl.BlockSpec((QA, C), lambda i: (i, 0)),
                pl.BlockSpec((QA, K), lambda i: (i, 0)),
                pl.BlockSpec((QA, C), lambda i: (i, 0)),
                wspec1, bspec1, wspec1,
                pl.BlockSpec((C, FF), lambda i: (0, 0)),
                pl.BlockSpec((1, FF), lambda i: (0, 0)),
                pl.BlockSpec((FF, C), lambda i: (0, 0)),
                bspec1, bspec1, bspec1, bspec1, bspec1,
            ],
            out_specs=pl.BlockSpec((QA, C), lambda i: (i, 0)),
            out_shape=jax.ShapeDtypeStruct((B * HN, C), f32),
        )(gk, sl(pq), sl(ad2), msk_h.reshape(B * HN, K), sl(q_feat),
          W_pos2, _row2(b_pos2), W_attn, W_ff1, _row2(b_ff1), W_ff2,
          _row2(b_ff2), _row2(ln1_g), _row2(ln1_b), _row2(ln2_g),
          _row2(ln2_b))
        return out_h.reshape(B, HN, C)

    idx0, msk0 = knn_half(0)
    gk0 = gather_half(idx0)
    idx1, msk1 = knn_half(1)
    gk1 = gather_half(idx1)
    out0 = attn_half(0, gk0, msk0)
    out1 = attn_half(1, gk1, msk1)
    return jnp.concatenate([out0, out1], axis=1)
